# baseline (speedup 1.0000x reference)
import jax
import jax.numpy as jnp
from jax import lax
from jax.experimental import pallas as pl

_NU = 0.1


def _tc_body(xt_ref, c_ref, r_ref, out_ref):
    xt = xt_ref[...]
    cm = c_ref[...]
    r = r_ref[...]
    B = xt.shape[1]
    xn2 = 0.25 * jnp.sum(xt * xt, axis=0, keepdims=True)
    ones_b = jnp.ones((1, B), jnp.float32)
    xt_aug = jnp.concatenate([xt, ones_b, xn2], axis=0)
    cn2 = jnp.sum(cm * cm, axis=1, keepdims=True)
    ones_k = jnp.ones((cm.shape[0], 1), jnp.float32)
    c_aug = jnp.concatenate([cm, cn2, ones_k], axis=1)
    d2 = lax.dot_general(xt_aug, c_aug, (((0,), (1,)), ((), ())),
                         preferred_element_type=jnp.float32)
    dmin = jnp.min(d2, axis=1, keepdims=True)
    r2 = r * r
    m = jnp.where(d2 == dmin, 1.0, 0.0)
    r2col = jnp.reshape(r2, (r2.shape[1], 1))
    r2sel = lax.dot_general(m, r2col, (((1,), (0,)), ((), ())),
                            preferred_element_type=jnp.float32)
    scores = dmin[:, 0] - r2sel[:, 0]
    total = jnp.sum(jnp.maximum(scores, 0.0))
    loss = jnp.mean(r2) + (1.0 / _NU) * (total / B)
    out_ref[...] = jnp.reshape(loss, (1, 1))


def kernel(input, c, R):
    B, D = input.shape
    K = c.shape[0]
    out = pl.pallas_call(
        _tc_body,
        grid=(1,),
        in_specs=[
            pl.BlockSpec((D, B), lambda i: (0, 0)),
            pl.BlockSpec((K, D), lambda i: (0, 0)),
            pl.BlockSpec((1, K), lambda i: (0, 0)),
        ],
        out_specs=pl.BlockSpec((1, 1), lambda i: (0, 0)),
        out_shape=jax.ShapeDtypeStruct((1, 1), jnp.float32),
    )((-2.0 * input).T, c, R.reshape(1, -1))
    return out[0, 0]


# speedup vs baseline: 1.1810x; 1.1810x over previous
import functools

import jax
import jax.numpy as jnp
from jax import lax
from jax.experimental import pallas as pl
from jax.experimental.pallas import tpu as pltpu

_NU = 0.1
_NSTEPS = 2


def _tc_body(xt_ref, c_ref, r_ref, out_ref, acc_ref, *, nsteps):
    i = pl.program_id(0)
    xt = xt_ref[...]
    cm = c_ref[...]
    r = r_ref[...]
    BB = xt.shape[1]
    xn2 = 0.25 * jnp.sum(xt * xt, axis=0, keepdims=True)
    ones_b = jnp.ones((1, BB), jnp.float32)
    xt_aug = jnp.concatenate([xt, ones_b, xn2], axis=0)
    cn2 = jnp.sum(cm * cm, axis=1, keepdims=True)
    ones_k = jnp.ones((cm.shape[0], 1), jnp.float32)
    c_aug = jnp.concatenate([cm, cn2, ones_k], axis=1)
    d2 = lax.dot_general(xt_aug, c_aug, (((0,), (1,)), ((), ())),
                         preferred_element_type=jnp.float32)
    dmin = jnp.min(d2, axis=1, keepdims=True)
    r2 = r * r
    r2sel = jnp.max(jnp.where(d2 == dmin, r2, -1.0), axis=1)
    scores = dmin[:, 0] - r2sel
    partial = jnp.sum(jnp.maximum(scores, 0.0))

    @pl.when(i == 0)
    def _():
        acc_ref[0] = 0.0

    acc_ref[0] += partial

    @pl.when(i == nsteps - 1)
    def _():
        loss = jnp.mean(r2) + (1.0 / _NU) * (acc_ref[0] / (nsteps * BB))
        out_ref[...] = jnp.reshape(loss, (1, 1))


def kernel(input, c, R):
    B, D = input.shape
    K = c.shape[0]
    bb = B // _NSTEPS
    out = pl.pallas_call(
        functools.partial(_tc_body, nsteps=_NSTEPS),
        grid=(_NSTEPS,),
        in_specs=[
            pl.BlockSpec((D, bb), lambda i: (0, i)),
            pl.BlockSpec((K, D), lambda i: (0, 0)),
            pl.BlockSpec((1, K), lambda i: (0, 0)),
        ],
        out_specs=pl.BlockSpec((1, 1), lambda i: (0, 0)),
        out_shape=jax.ShapeDtypeStruct((1, 1), jnp.float32),
        scratch_shapes=[pltpu.SMEM((1,), jnp.float32)],
    )((-2.0 * input).T, c, R.reshape(1, -1))
    return out[0, 0]


# speedup vs baseline: 1.2265x; 1.0385x over previous
import jax
import jax.numpy as jnp
from jax import lax
from jax.experimental import pallas as pl

_NU = 0.1


def _tc_body(xt_ref, c_ref, r_ref, out_ref):
    xt = xt_ref[...]
    cm = c_ref[...]
    r = r_ref[...]
    B = xt.shape[1]
    xn2 = 0.25 * jnp.sum(xt * xt, axis=0, keepdims=True)
    ones_b = jnp.ones((1, B), jnp.float32)
    xt_aug = jnp.concatenate([xt, ones_b, xn2], axis=0)
    cn2 = jnp.sum(cm * cm, axis=1, keepdims=True)
    ones_k = jnp.ones((cm.shape[0], 1), jnp.float32)
    c_aug = jnp.concatenate([cm, cn2, ones_k], axis=1)
    d2 = lax.dot_general(xt_aug, c_aug, (((0,), (1,)), ((), ())),
                         preferred_element_type=jnp.float32)
    dmin = jnp.min(d2, axis=1, keepdims=True)
    r2 = r * r
    r2sel = jnp.max(jnp.where(d2 == dmin, r2, -1.0), axis=1)
    scores = dmin[:, 0] - r2sel
    total = jnp.sum(jnp.maximum(scores, 0.0))
    loss = jnp.mean(r2) + (1.0 / _NU) * (total / B)
    out_ref[...] = jnp.reshape(loss, (1, 1))


def kernel(input, c, R):
    B, D = input.shape
    K = c.shape[0]
    out = pl.pallas_call(
        _tc_body,
        grid=(1,),
        in_specs=[
            pl.BlockSpec((D, B), lambda i: (0, 0)),
            pl.BlockSpec((K, D), lambda i: (0, 0)),
            pl.BlockSpec((1, K), lambda i: (0, 0)),
        ],
        out_specs=pl.BlockSpec((1, 1), lambda i: (0, 0)),
        out_shape=jax.ShapeDtypeStruct((1, 1), jnp.float32),
    )((-2.0 * input).T, c, R.reshape(1, -1))
    return out[0, 0]
